# baseline (speedup 1.0000x reference)
import functools

import jax
import jax.numpy as jnp
from jax.experimental import pallas as pl
from jax.experimental.pallas import tpu as pltpu

_NB = 4
_D = 3


def _se_pipe_kernel(x_hbm, w1t_ref, b1_ref, w2t_ref, b2_ref, o_hbm,
                    in_buf, out_buf, in_sems, out_sems, *, nb, nt, inv_hw):
    def in_copy(t):
        return pltpu.make_async_copy(
            x_hbm.at[pl.ds(t * nb, nb)], in_buf.at[t % _D], in_sems.at[t % _D])

    def out_copy(t):
        return pltpu.make_async_copy(
            out_buf.at[t % _D], o_hbm.at[pl.ds(t * nb, nb)], out_sems.at[t % _D])

    for d in range(min(_D, nt)):
        in_copy(d).start()

    for t in range(nt):
        slot = t % _D
        in_copy(t).wait()
        if t >= _D:
            out_copy(t - _D).wait()
        xv = in_buf[slot]
        pooled = jnp.sum(xv, axis=-1) * inv_hw
        h = jnp.maximum(
            jnp.dot(pooled, w1t_ref[...],
                    preferred_element_type=jnp.float32) + b1_ref[...], 0.0)
        s = jax.nn.sigmoid(
            jnp.dot(h, w2t_ref[...],
                    preferred_element_type=jnp.float32) + b2_ref[...])
        out_buf[slot] = xv * s[:, :, None]
        out_copy(t).start()
        if t + _D < nt:
            in_copy(t + _D).start()

    for t in range(max(nt - _D, 0), nt):
        out_copy(t).wait()


def kernel(x, w1, b1, w2, b2):
    N, C, H, W = x.shape
    Ch = w1.shape[0]
    HW = H * W
    x_flat = x.reshape(N, C, HW)
    w1t = w1.T
    w2t = w2.T
    b1r = b1.reshape(1, Ch)
    b2r = b2.reshape(1, C)

    nb = _NB
    nt = N // nb
    out_flat = pl.pallas_call(
        functools.partial(_se_pipe_kernel, nb=nb, nt=nt, inv_hw=1.0 / HW),
        out_shape=jax.ShapeDtypeStruct((N, C, HW), x.dtype),
        in_specs=[
            pl.BlockSpec(memory_space=pl.ANY),
            pl.BlockSpec((C, Ch), lambda: (0, 0)),
            pl.BlockSpec((1, Ch), lambda: (0, 0)),
            pl.BlockSpec((Ch, C), lambda: (0, 0)),
            pl.BlockSpec((1, C), lambda: (0, 0)),
        ],
        out_specs=pl.BlockSpec(memory_space=pl.ANY),
        scratch_shapes=[
            pltpu.VMEM((_D, nb, C, HW), jnp.float32),
            pltpu.VMEM((_D, nb, C, HW), jnp.float32),
            pltpu.SemaphoreType.DMA((_D,)),
            pltpu.SemaphoreType.DMA((_D,)),
        ],
        compiler_params=pltpu.CompilerParams(vmem_limit_bytes=60 << 20),
        cost_estimate=pl.CostEstimate(
            flops=int(4 * N * C * Ch + 2 * N * C * HW),
            transcendentals=int(N * C),
            bytes_accessed=int(2 * N * C * HW * 4),
        ),
    )(x_flat, w1t, b1r, w2t, b2r)
    return out_flat.reshape(N, C, H, W)
